# baseline (speedup 1.0000x reference)
import functools

import jax
import jax.numpy as jnp
from jax.experimental import pallas as pl
from jax.experimental.pallas import tpu as pltpu

_B = 4
_E = 3
_PREC = jax.lax.Precision.HIGHEST


def _dot(a, b):
    return jnp.dot(a, b, preferred_element_type=jnp.float32, precision=_PREC)


_GCH = 16


def _gate_kernel(x_hbm, wg_ref, beta_ref, fct_ref, fcb_ref, logits_ref,
                 xbuf, f_ref, sems):
    i = pl.program_id(0)
    nchunk = 192 // _GCH

    def _copy(j):
        return pltpu.make_async_copy(
            x_hbm.at[i, pl.ds(_GCH * j, _GCH + 3)], xbuf.at[j % 2], sems.at[j % 2])

    _copy(0).start()
    neg = jnp.float32(-3e38)
    f_ref[0:1, :, :] = jnp.full((1, 192, 64), neg, jnp.float32)
    f_ref[193:194, :, :] = jnp.full((1, 192, 64), neg, jnp.float32)
    beta = beta_ref[0][None, :]
    for j in range(nchunk):
        r0 = j * _GCH
        _copy(j).wait()
        if j + 1 < nchunk:
            _copy(j + 1).start()
        acc = jnp.zeros((_GCH * 192, 64), jnp.float32)
        for ty in range(4):
            for tx in range(4):
                tt = ty * 4 + tx
                A = xbuf[j % 2, ty:ty + _GCH, tx:tx + 192, :].reshape(_GCH * 192, 60)
                acc = acc + _dot(A, wg_ref[tt * 60:(tt + 1) * 60, :])
        f_ref[1 + r0:1 + r0 + _GCH, :, :] = (
            jnp.maximum(acc + beta, 0.0).reshape(_GCH, 192, 64))
    feat = jnp.zeros((1, 64), jnp.float32)
    for pc in range(12):
        b0 = 16 * pc
        MA = f_ref[b0:b0 + 16, :, :].reshape(8, 2, 192, 64).max(axis=1)
        OB = f_ref[b0 + 2:b0 + 18, :, :].reshape(8, 2, 192, 64)[:, 0]
        W8 = jnp.maximum(MA, OB)
        W8r = W8.reshape(8, 96, 2, 64)
        Mc = W8r.max(axis=2)
        Oc = W8r[:, :, 1, :]
        OcS = jnp.concatenate(
            [jnp.full((8, 1, 64), neg, jnp.float32), Oc[:, :-1]], axis=1)
        pooled = jnp.maximum(Mc, OcS).reshape(8 * 96, 64)
        feat = feat + jnp.sum(pooled, axis=0, keepdims=True)
    feat = feat * (1.0 / 9216.0)
    logits_ref[...] = (_dot(feat, fct_ref[...]) + fcb_ref[...]).reshape(1, 1, 128)


def _route_kernel(l_ref, idx_ref, aux_ref):
    L = l_ref[...].reshape(_B, 128)
    l0, l1, l2 = L[:, 0:1], L[:, 1:2], L[:, 2:3]
    i32 = lambda m: m.astype(jnp.int32)
    r0 = i32(l1 > l0) + i32(l2 > l0)
    r1 = i32(l0 >= l1) + i32(l2 > l1)
    r2 = i32(l0 >= l2) + i32(l1 >= l2)
    k0, k1, k2 = r0 < 2, r1 < 2, r2 < 2
    e_lo = jnp.where(k0, 0, 1).astype(jnp.int32)
    e_hi = jnp.where(k2, 2, 1).astype(jnp.int32)
    idx_ref[...] = jnp.concatenate([e_lo, e_hi], axis=1)
    m = jnp.maximum(jnp.maximum(l0, l1), l2)
    p0, p1, p2 = jnp.exp(l0 - m), jnp.exp(l1 - m), jnp.exp(l2 - m)
    s = p0 + p1 + p2
    f32 = lambda m_: m_.astype(jnp.float32)
    quarter = 0.25
    d0 = jnp.sum(f32(k0), axis=0, keepdims=True) * quarter
    d1 = jnp.sum(f32(k1), axis=0, keepdims=True) * quarter
    d2 = jnp.sum(f32(k2), axis=0, keepdims=True) * quarter
    q0 = jnp.sum(p0 / s, axis=0, keepdims=True) * quarter
    q1 = jnp.sum(p1 / s, axis=0, keepdims=True) * quarter
    q2 = jnp.sum(p2 / s, axis=0, keepdims=True) * quarter
    aux_ref[...] = 0.03 * (d0 * q0 + d1 * q1 + d2 * q2)


_ECH = 16


def _expert_kernel(idx_ref, x_ref, w1_ref, b1_ref, w2_ref, b2_ref,
                   wh_ref, bh_ref, out_ref, c1_ref):
    k = pl.program_id(1)
    b1 = b1_ref[0]
    b2 = b2_ref[0]
    bh = bh_ref[0]
    c1_ref[0:1, :, :] = jnp.zeros((1, 98, 64), jnp.float32)
    c1_ref[97:98, :, :] = jnp.zeros((1, 98, 64), jnp.float32)
    for j in range(96 // _ECH):
        r0 = j * _ECH
        acc = jnp.zeros((_ECH * 96, 64), jnp.float32)
        for dy in range(2):
            for dx in range(2):
                tt = dy * 2 + dx
                A = x_ref[0, r0 + dy:r0 + dy + _ECH, dx:dx + 96, :].reshape(_ECH * 96, 240)
                acc = acc + _dot(A, w1_ref[0, tt * 240:(tt + 1) * 240, :])
        h1 = jnp.maximum(acc + b1, 0.0).reshape(_ECH, 96, 64)
        c1_ref[1 + r0:1 + r0 + _ECH, 1:97, :] = h1
        c1_ref[1 + r0:1 + r0 + _ECH, 0:1, :] = jnp.zeros((_ECH, 1, 64), jnp.float32)
        c1_ref[1 + r0:1 + r0 + _ECH, 97:98, :] = jnp.zeros((_ECH, 1, 64), jnp.float32)
    for j in range(96 // _ECH):
        r0 = j * _ECH
        acc2 = jnp.zeros((_ECH * 96, 128), jnp.float32)
        for ty in range(3):
            for tx in range(3):
                tt = ty * 3 + tx
                A2 = c1_ref[r0 + ty:r0 + ty + _ECH, tx:tx + 96, :].reshape(_ECH * 96, 64)
                acc2 = acc2 + _dot(A2, w2_ref[0, tt * 64:(tt + 1) * 64, :])
        h2 = jnp.maximum(acc2 + b2, 0.0)
        res = ((_dot(h2, wh_ref[0]) + bh) * 0.5).reshape(_ECH, 96, 8)

        @pl.when(k == 0)
        def _():
            out_ref[0, r0:r0 + _ECH] = res

        @pl.when(k == 1)
        def _():
            out_ref[0, r0:r0 + _ECH] = out_ref[0, r0:r0 + _ECH] + res


def kernel(x, gate_conv_w, gate_bn_gamma, gate_bn_beta, gate_fc_w, gate_fc_b,
           exp_conv1_w, exp_conv1_b, exp_conv2_w, exp_conv2_b, exp_hm_w,
           exp_hm_b, exp_wh_w, exp_wh_b, exp_reg_w, exp_reg_b):
    b, c, t, h, w = x.shape
    xr = x.reshape(b, c * t, h, w)

    scale = gate_bn_gamma / jnp.sqrt(1.0 + 1e-5)
    wfold = gate_conv_w * scale[:, None, None, None]
    Wgp = jnp.pad(wfold, ((0, 0), (0, 0), (1, 0), (1, 0)))
    Wg = Wgp.reshape(64, 15, 4, 2, 4, 2).transpose(2, 4, 3, 5, 1, 0).reshape(960, 64)
    beta2 = gate_bn_beta.reshape(1, 64)
    fct = jnp.pad(gate_fc_w.T, ((0, 0), (0, 125)))
    fcb = jnp.pad(gate_fc_b, ((0, 125))).reshape(1, 128)

    xg = xr.reshape(b, 15, 192, 2, 192, 2).transpose(0, 2, 4, 3, 5, 1).reshape(b, 192, 192, 60)
    xg_pad = jnp.pad(xg, ((0, 0), (2, 1), (2, 1), (0, 0)))

    xe = xr.reshape(b, 15, 96, 4, 96, 4).transpose(0, 2, 4, 3, 5, 1).reshape(b, 96, 96, 240)
    xe_pad = jnp.pad(xe, ((0, 0), (1, 0), (1, 0), (0, 0)))

    W1p = jnp.pad(exp_conv1_w, ((0, 0), (0, 0), (0, 0), (1, 0), (1, 0)))
    W1s = W1p.reshape(_E, 64, 15, 2, 4, 2, 4).transpose(0, 3, 5, 4, 6, 2, 1).reshape(_E, 960, 64)
    W2s = exp_conv2_w.transpose(0, 3, 4, 2, 1).reshape(_E, 576, 128)
    Whead = jnp.concatenate([exp_hm_w, exp_wh_w, exp_reg_w], axis=1)
    Whead = jnp.pad(Whead.reshape(_E, 5, 128).transpose(0, 2, 1), ((0, 0), (0, 0), (0, 3)))
    bhead = jnp.pad(jnp.concatenate([exp_hm_b, exp_wh_b, exp_reg_b], axis=1), ((0, 0), (0, 3)))

    logits = pl.pallas_call(
        _gate_kernel,
        grid=(b,),
        in_specs=[
            pl.BlockSpec(memory_space=pl.ANY),
            pl.BlockSpec((960, 64), lambda i: (0, 0)),
            pl.BlockSpec((1, 64), lambda i: (0, 0)),
            pl.BlockSpec((64, 128), lambda i: (0, 0)),
            pl.BlockSpec((1, 128), lambda i: (0, 0)),
        ],
        out_specs=pl.BlockSpec((1, 1, 128), lambda i: (i, 0, 0)),
        out_shape=jax.ShapeDtypeStruct((b, 1, 128), jnp.float32),
        scratch_shapes=[
            pltpu.VMEM((2, _GCH + 3, 195, 60), jnp.float32),
            pltpu.VMEM((194, 192, 64), jnp.float32),
            pltpu.SemaphoreType.DMA((2,)),
        ],
    )(xg_pad, Wg, beta2, fct, fcb)

    idx, aux = pl.pallas_call(
        _route_kernel,
        out_shape=(jax.ShapeDtypeStruct((b, 2), jnp.int32),
                   jax.ShapeDtypeStruct((1, 1), jnp.float32)),
    )(logits)
    idx_flat = idx.reshape(b * 2)

    grid_spec = pltpu.PrefetchScalarGridSpec(
        num_scalar_prefetch=1,
        grid=(b, 2),
        in_specs=[
            pl.BlockSpec((1, 97, 97, 240), lambda i, k, idx_r: (i, 0, 0, 0)),
            pl.BlockSpec((1, 960, 64), lambda i, k, idx_r: (idx_r[2 * i + k], 0, 0)),
            pl.BlockSpec((1, 1, 64), lambda i, k, idx_r: (idx_r[2 * i + k], 0, 0)),
            pl.BlockSpec((1, 576, 128), lambda i, k, idx_r: (idx_r[2 * i + k], 0, 0)),
            pl.BlockSpec((1, 1, 128), lambda i, k, idx_r: (idx_r[2 * i + k], 0, 0)),
            pl.BlockSpec((1, 128, 8), lambda i, k, idx_r: (idx_r[2 * i + k], 0, 0)),
            pl.BlockSpec((1, 1, 8), lambda i, k, idx_r: (idx_r[2 * i + k], 0, 0)),
        ],
        out_specs=pl.BlockSpec((1, 96, 96, 8), lambda i, k, idx_r: (i, 0, 0, 0)),
        scratch_shapes=[pltpu.VMEM((98, 98, 64), jnp.float32)],
    )
    out = pl.pallas_call(
        _expert_kernel,
        grid_spec=grid_spec,
        out_shape=jax.ShapeDtypeStruct((b, 96, 96, 8), jnp.float32),
    )(idx_flat, xe_pad, W1s, exp_conv1_b.reshape(_E, 1, 64), W2s,
      exp_conv2_b.reshape(_E, 1, 128), Whead, bhead.reshape(_E, 1, 8))

    o = out.transpose(0, 3, 1, 2)
    hm = o[:, 0:1]
    wh_o = o[:, 1:3]
    reg = o[:, 3:5]
    return hm, wh_o, reg, aux[0, 0]


# speedup vs baseline: 1.1481x; 1.1481x over previous
import functools

import jax
import jax.numpy as jnp
from jax.experimental import pallas as pl
from jax.experimental.pallas import tpu as pltpu

_B = 4
_E = 3


def _dot(a, b):
    return jnp.dot(a, b, preferred_element_type=jnp.float32,
                   precision=jax.lax.Precision.HIGHEST)


def _dot_fast(a, b):
    return jnp.dot(a, b, preferred_element_type=jnp.float32,
                   precision=jax.lax.Precision.DEFAULT)


_GCH = 16


def _gate_kernel(x_hbm, wg_ref, beta_ref, fct_ref, fcb_ref, logits_ref,
                 xbuf, f_ref, sems):
    i = pl.program_id(0)
    nchunk = 192 // _GCH

    def _copy(j):
        return pltpu.make_async_copy(
            x_hbm.at[i, pl.ds(_GCH * j, _GCH + 3)], xbuf.at[j % 2], sems.at[j % 2])

    _copy(0).start()
    neg = jnp.float32(-3e38)
    f_ref[0:1, :, :] = jnp.full((1, 192, 64), neg, jnp.float32)
    f_ref[193:194, :, :] = jnp.full((1, 192, 64), neg, jnp.float32)
    beta = beta_ref[0][None, :]
    for j in range(nchunk):
        r0 = j * _GCH
        _copy(j).wait()
        if j + 1 < nchunk:
            _copy(j + 1).start()
        acc = jnp.zeros((_GCH * 192, 64), jnp.float32)
        for ty in range(4):
            for tx in range(4):
                tt = ty * 4 + tx
                A = xbuf[j % 2, ty:ty + _GCH, tx:tx + 192, :].reshape(_GCH * 192, 60)
                acc = acc + _dot(A, wg_ref[tt * 60:(tt + 1) * 60, :])
        f_ref[1 + r0:1 + r0 + _GCH, :, :] = (
            jnp.maximum(acc + beta, 0.0).reshape(_GCH, 192, 64))
    feat = jnp.zeros((1, 64), jnp.float32)
    for pc in range(12):
        b0 = 16 * pc
        MA = f_ref[b0:b0 + 16, :, :].reshape(8, 2, 192, 64).max(axis=1)
        OB = f_ref[b0 + 2:b0 + 18, :, :].reshape(8, 2, 192, 64)[:, 0]
        W8 = jnp.maximum(MA, OB)
        W8r = W8.reshape(8, 96, 2, 64)
        Mc = W8r.max(axis=2)
        Oc = W8r[:, :, 1, :]
        OcS = jnp.concatenate(
            [jnp.full((8, 1, 64), neg, jnp.float32), Oc[:, :-1]], axis=1)
        pooled = jnp.maximum(Mc, OcS).reshape(8 * 96, 64)
        feat = feat + jnp.sum(pooled, axis=0, keepdims=True)
    feat = feat * (1.0 / 9216.0)
    logits_ref[...] = (_dot(feat, fct_ref[...]) + fcb_ref[...]).reshape(1, 1, 128)


def _route_kernel(l_ref, idx_ref, aux_ref):
    L = l_ref[...].reshape(_B, 128)
    l0, l1, l2 = L[:, 0:1], L[:, 1:2], L[:, 2:3]
    i32 = lambda m: m.astype(jnp.int32)
    r0 = i32(l1 > l0) + i32(l2 > l0)
    r1 = i32(l0 >= l1) + i32(l2 > l1)
    r2 = i32(l0 >= l2) + i32(l1 >= l2)
    k0, k1, k2 = r0 < 2, r1 < 2, r2 < 2
    e_lo = jnp.where(k0, 0, 1).astype(jnp.int32)
    e_hi = jnp.where(k2, 2, 1).astype(jnp.int32)
    idx_ref[...] = jnp.concatenate([e_lo, e_hi], axis=1)
    m = jnp.maximum(jnp.maximum(l0, l1), l2)
    p0, p1, p2 = jnp.exp(l0 - m), jnp.exp(l1 - m), jnp.exp(l2 - m)
    s = p0 + p1 + p2
    f32 = lambda m_: m_.astype(jnp.float32)
    quarter = 0.25
    d0 = jnp.sum(f32(k0), axis=0, keepdims=True) * quarter
    d1 = jnp.sum(f32(k1), axis=0, keepdims=True) * quarter
    d2 = jnp.sum(f32(k2), axis=0, keepdims=True) * quarter
    q0 = jnp.sum(p0 / s, axis=0, keepdims=True) * quarter
    q1 = jnp.sum(p1 / s, axis=0, keepdims=True) * quarter
    q2 = jnp.sum(p2 / s, axis=0, keepdims=True) * quarter
    aux_ref[...] = 0.03 * (d0 * q0 + d1 * q1 + d2 * q2)


_ECH = 16


def _expert_kernel(idx_ref, x_ref, w1_ref, b1_ref, w2_ref, b2_ref,
                   wh_ref, bh_ref, out_ref, c1_ref):
    k = pl.program_id(1)
    b1 = b1_ref[0]
    b2 = b2_ref[0]
    bh = bh_ref[0]
    c1_ref[0:1, :, :] = jnp.zeros((1, 98, 64), jnp.float32)
    c1_ref[97:98, :, :] = jnp.zeros((1, 98, 64), jnp.float32)
    for j in range(96 // _ECH):
        r0 = j * _ECH
        acc = jnp.zeros((_ECH * 96, 64), jnp.float32)
        for dy in range(2):
            for dx in range(2):
                tt = dy * 2 + dx
                A = x_ref[0, r0 + dy:r0 + dy + _ECH, dx:dx + 96, :].reshape(_ECH * 96, 240)
                acc = acc + _dot_fast(A, w1_ref[0, tt * 240:(tt + 1) * 240, :])
        h1 = jnp.maximum(acc + b1, 0.0).reshape(_ECH, 96, 64)
        c1_ref[1 + r0:1 + r0 + _ECH, 1:97, :] = h1
        c1_ref[1 + r0:1 + r0 + _ECH, 0:1, :] = jnp.zeros((_ECH, 1, 64), jnp.float32)
        c1_ref[1 + r0:1 + r0 + _ECH, 97:98, :] = jnp.zeros((_ECH, 1, 64), jnp.float32)
    for j in range(96 // _ECH):
        r0 = j * _ECH
        acc2 = jnp.zeros((_ECH * 96, 128), jnp.float32)
        for ty in range(3):
            for tx in range(3):
                tt = ty * 3 + tx
                A2 = c1_ref[r0 + ty:r0 + ty + _ECH, tx:tx + 96, :].reshape(_ECH * 96, 64)
                acc2 = acc2 + _dot_fast(A2, w2_ref[0, tt * 64:(tt + 1) * 64, :])
        h2 = jnp.maximum(acc2 + b2, 0.0)
        res = ((_dot_fast(h2, wh_ref[0]) + bh) * 0.5).reshape(_ECH, 96, 8)

        @pl.when(k == 0)
        def _():
            out_ref[0, r0:r0 + _ECH] = res

        @pl.when(k == 1)
        def _():
            out_ref[0, r0:r0 + _ECH] = out_ref[0, r0:r0 + _ECH] + res


def kernel(x, gate_conv_w, gate_bn_gamma, gate_bn_beta, gate_fc_w, gate_fc_b,
           exp_conv1_w, exp_conv1_b, exp_conv2_w, exp_conv2_b, exp_hm_w,
           exp_hm_b, exp_wh_w, exp_wh_b, exp_reg_w, exp_reg_b):
    b, c, t, h, w = x.shape
    xr = x.reshape(b, c * t, h, w)

    scale = gate_bn_gamma / jnp.sqrt(1.0 + 1e-5)
    wfold = gate_conv_w * scale[:, None, None, None]
    Wgp = jnp.pad(wfold, ((0, 0), (0, 0), (1, 0), (1, 0)))
    Wg = Wgp.reshape(64, 15, 4, 2, 4, 2).transpose(2, 4, 3, 5, 1, 0).reshape(960, 64)
    beta2 = gate_bn_beta.reshape(1, 64)
    fct = jnp.pad(gate_fc_w.T, ((0, 0), (0, 125)))
    fcb = jnp.pad(gate_fc_b, ((0, 125))).reshape(1, 128)

    xg = xr.reshape(b, 15, 192, 2, 192, 2).transpose(0, 2, 4, 3, 5, 1).reshape(b, 192, 192, 60)
    xg_pad = jnp.pad(xg, ((0, 0), (2, 1), (2, 1), (0, 0)))

    xe = xr.reshape(b, 15, 96, 4, 96, 4).transpose(0, 2, 4, 3, 5, 1).reshape(b, 96, 96, 240)
    xe_pad = jnp.pad(xe, ((0, 0), (1, 0), (1, 0), (0, 0)))

    W1p = jnp.pad(exp_conv1_w, ((0, 0), (0, 0), (0, 0), (1, 0), (1, 0)))
    W1s = W1p.reshape(_E, 64, 15, 2, 4, 2, 4).transpose(0, 3, 5, 4, 6, 2, 1).reshape(_E, 960, 64)
    W2s = exp_conv2_w.transpose(0, 3, 4, 2, 1).reshape(_E, 576, 128)
    Whead = jnp.concatenate([exp_hm_w, exp_wh_w, exp_reg_w], axis=1)
    Whead = jnp.pad(Whead.reshape(_E, 5, 128).transpose(0, 2, 1), ((0, 0), (0, 0), (0, 3)))
    bhead = jnp.pad(jnp.concatenate([exp_hm_b, exp_wh_b, exp_reg_b], axis=1), ((0, 0), (0, 3)))

    logits = pl.pallas_call(
        _gate_kernel,
        grid=(b,),
        in_specs=[
            pl.BlockSpec(memory_space=pl.ANY),
            pl.BlockSpec((960, 64), lambda i: (0, 0)),
            pl.BlockSpec((1, 64), lambda i: (0, 0)),
            pl.BlockSpec((64, 128), lambda i: (0, 0)),
            pl.BlockSpec((1, 128), lambda i: (0, 0)),
        ],
        out_specs=pl.BlockSpec((1, 1, 128), lambda i: (i, 0, 0)),
        out_shape=jax.ShapeDtypeStruct((b, 1, 128), jnp.float32),
        scratch_shapes=[
            pltpu.VMEM((2, _GCH + 3, 195, 60), jnp.float32),
            pltpu.VMEM((194, 192, 64), jnp.float32),
            pltpu.SemaphoreType.DMA((2,)),
        ],
    )(xg_pad, Wg, beta2, fct, fcb)

    idx, aux = pl.pallas_call(
        _route_kernel,
        out_shape=(jax.ShapeDtypeStruct((b, 2), jnp.int32),
                   jax.ShapeDtypeStruct((1, 1), jnp.float32)),
    )(logits)
    idx_flat = idx.reshape(b * 2)

    grid_spec = pltpu.PrefetchScalarGridSpec(
        num_scalar_prefetch=1,
        grid=(b, 2),
        in_specs=[
            pl.BlockSpec((1, 97, 97, 240), lambda i, k, idx_r: (i, 0, 0, 0)),
            pl.BlockSpec((1, 960, 64), lambda i, k, idx_r: (idx_r[2 * i + k], 0, 0)),
            pl.BlockSpec((1, 1, 64), lambda i, k, idx_r: (idx_r[2 * i + k], 0, 0)),
            pl.BlockSpec((1, 576, 128), lambda i, k, idx_r: (idx_r[2 * i + k], 0, 0)),
            pl.BlockSpec((1, 1, 128), lambda i, k, idx_r: (idx_r[2 * i + k], 0, 0)),
            pl.BlockSpec((1, 128, 8), lambda i, k, idx_r: (idx_r[2 * i + k], 0, 0)),
            pl.BlockSpec((1, 1, 8), lambda i, k, idx_r: (idx_r[2 * i + k], 0, 0)),
        ],
        out_specs=pl.BlockSpec((1, 96, 96, 8), lambda i, k, idx_r: (i, 0, 0, 0)),
        scratch_shapes=[pltpu.VMEM((98, 98, 64), jnp.float32)],
    )
    out = pl.pallas_call(
        _expert_kernel,
        grid_spec=grid_spec,
        out_shape=jax.ShapeDtypeStruct((b, 96, 96, 8), jnp.float32),
    )(idx_flat, xe_pad, W1s, exp_conv1_b.reshape(_E, 1, 64), W2s,
      exp_conv2_b.reshape(_E, 1, 128), Whead, bhead.reshape(_E, 1, 8))

    o = out.transpose(0, 3, 1, 2)
    hm = o[:, 0:1]
    wh_o = o[:, 1:3]
    reg = o[:, 3:5]
    return hm, wh_o, reg, aux[0, 0]


# speedup vs baseline: 1.3710x; 1.1942x over previous
import functools

import jax
import jax.numpy as jnp
from jax.experimental import pallas as pl
from jax.experimental.pallas import tpu as pltpu

_B = 4
_E = 3


def _dot(a, b):
    return jnp.dot(a, b, preferred_element_type=jnp.float32,
                   precision=jax.lax.Precision.HIGHEST)


def _dot_fast(a, b):
    return jnp.dot(a, b, preferred_element_type=jnp.float32,
                   precision=jax.lax.Precision.DEFAULT)


_GCH = 16


def _gate_kernel(x_hbm, wg_ref, beta_ref, fct_ref, fcb_ref, logits_ref,
                 xbuf, f_ref, sems):
    i = pl.program_id(0)
    nchunk = 192 // _GCH

    def _copy(j):
        return pltpu.make_async_copy(
            x_hbm.at[i, pl.ds(_GCH * j, _GCH + 3)], xbuf.at[j % 2], sems.at[j % 2])

    _copy(0).start()
    neg = jnp.float32(-3e38)
    f_ref[0:1, :, :] = jnp.full((1, 192, 64), neg, jnp.float32)
    f_ref[193:194, :, :] = jnp.full((1, 192, 64), neg, jnp.float32)
    beta = beta_ref[0][None, :]
    for j in range(nchunk):
        r0 = j * _GCH
        _copy(j).wait()
        if j + 1 < nchunk:
            _copy(j + 1).start()
        acc = jnp.zeros((_GCH * 192, 64), jnp.float32)
        for ty in range(4):
            for tx in range(4):
                tt = ty * 4 + tx
                A = xbuf[j % 2, ty:ty + _GCH, tx:tx + 192, :].reshape(_GCH * 192, 60)
                acc = acc + _dot(A, wg_ref[tt * 60:(tt + 1) * 60, :])
        f_ref[1 + r0:1 + r0 + _GCH, :, :] = (
            jnp.maximum(acc + beta, 0.0).reshape(_GCH, 192, 64))
    feat = jnp.zeros((1, 64), jnp.float32)
    for pc in range(12):
        b0 = 16 * pc
        MA = f_ref[b0:b0 + 16, :, :].reshape(8, 2, 192, 64).max(axis=1)
        OB = f_ref[b0 + 2:b0 + 18, :, :].reshape(8, 2, 192, 64)[:, 0]
        W8 = jnp.maximum(MA, OB)
        W8r = W8.reshape(8, 96, 2, 64)
        Mc = W8r.max(axis=2)
        Oc = W8r[:, :, 1, :]
        OcS = jnp.concatenate(
            [jnp.full((8, 1, 64), neg, jnp.float32), Oc[:, :-1]], axis=1)
        pooled = jnp.maximum(Mc, OcS).reshape(8 * 96, 64)
        feat = feat + jnp.sum(pooled, axis=0, keepdims=True)
    feat = feat * (1.0 / 9216.0)
    logits_ref[...] = (_dot(feat, fct_ref[...]) + fcb_ref[...]).reshape(1, 1, 128)


def _s2d4_kernel(x_ref, o_ref):
    X = x_ref[0]
    o_ref[0] = X.reshape(15, 2, 4, 96, 4).transpose(1, 3, 2, 4, 0).reshape(2, 96, 240)


def _s2d2_kernel(x_ref, o_ref):
    X = x_ref[0]
    o_ref[0] = X.reshape(15, 4, 2, 192, 2).transpose(1, 3, 2, 4, 0).reshape(4, 192, 60)


def _route_kernel(l_ref, idx_ref, aux_ref):
    L = l_ref[...].reshape(_B, 128)
    l0, l1, l2 = L[:, 0:1], L[:, 1:2], L[:, 2:3]
    i32 = lambda m: m.astype(jnp.int32)
    r0 = i32(l1 > l0) + i32(l2 > l0)
    r1 = i32(l0 >= l1) + i32(l2 > l1)
    r2 = i32(l0 >= l2) + i32(l1 >= l2)
    k0, k1, k2 = r0 < 2, r1 < 2, r2 < 2
    e_lo = jnp.where(k0, 0, 1).astype(jnp.int32)
    e_hi = jnp.where(k2, 2, 1).astype(jnp.int32)
    idx_ref[...] = jnp.concatenate([e_lo, e_hi], axis=1)
    m = jnp.maximum(jnp.maximum(l0, l1), l2)
    p0, p1, p2 = jnp.exp(l0 - m), jnp.exp(l1 - m), jnp.exp(l2 - m)
    s = p0 + p1 + p2
    f32 = lambda m_: m_.astype(jnp.float32)
    quarter = 0.25
    d0 = jnp.sum(f32(k0), axis=0, keepdims=True) * quarter
    d1 = jnp.sum(f32(k1), axis=0, keepdims=True) * quarter
    d2 = jnp.sum(f32(k2), axis=0, keepdims=True) * quarter
    q0 = jnp.sum(p0 / s, axis=0, keepdims=True) * quarter
    q1 = jnp.sum(p1 / s, axis=0, keepdims=True) * quarter
    q2 = jnp.sum(p2 / s, axis=0, keepdims=True) * quarter
    aux_ref[...] = 0.03 * (d0 * q0 + d1 * q1 + d2 * q2)


_ECH = 16


def _expert_kernel(idx_ref, x_ref, w1_ref, b1_ref, w2_ref, b2_ref,
                   wh_ref, bh_ref, out_ref, c1_ref):
    k = pl.program_id(1)
    b1 = b1_ref[0]
    b2 = b2_ref[0]
    bh = bh_ref[0]
    c1_ref[0:1, :, :] = jnp.zeros((1, 98, 64), jnp.float32)
    c1_ref[97:98, :, :] = jnp.zeros((1, 98, 64), jnp.float32)
    for j in range(96 // _ECH):
        r0 = j * _ECH
        acc = jnp.zeros((_ECH * 96, 64), jnp.float32)
        for dy in range(2):
            for dx in range(2):
                tt = dy * 2 + dx
                A = x_ref[0, r0 + dy:r0 + dy + _ECH, dx:dx + 96, :].reshape(_ECH * 96, 240)
                acc = acc + _dot_fast(A, w1_ref[0, tt * 240:(tt + 1) * 240, :])
        h1 = jnp.maximum(acc + b1, 0.0).reshape(_ECH, 96, 64)
        c1_ref[1 + r0:1 + r0 + _ECH, 1:97, :] = h1
        c1_ref[1 + r0:1 + r0 + _ECH, 0:1, :] = jnp.zeros((_ECH, 1, 64), jnp.float32)
        c1_ref[1 + r0:1 + r0 + _ECH, 97:98, :] = jnp.zeros((_ECH, 1, 64), jnp.float32)
    for j in range(96 // _ECH):
        r0 = j * _ECH
        acc2 = jnp.zeros((_ECH * 96, 128), jnp.float32)
        for ty in range(3):
            for tx in range(3):
                tt = ty * 3 + tx
                A2 = c1_ref[r0 + ty:r0 + ty + _ECH, tx:tx + 96, :].reshape(_ECH * 96, 64)
                acc2 = acc2 + _dot_fast(A2, w2_ref[0, tt * 64:(tt + 1) * 64, :])
        h2 = jnp.maximum(acc2 + b2, 0.0)
        res = ((_dot_fast(h2, wh_ref[0]) + bh) * 0.5).reshape(_ECH, 96, 8)

        @pl.when(k == 0)
        def _():
            out_ref[0, r0:r0 + _ECH] = res

        @pl.when(k == 1)
        def _():
            out_ref[0, r0:r0 + _ECH] = out_ref[0, r0:r0 + _ECH] + res


def kernel(x, gate_conv_w, gate_bn_gamma, gate_bn_beta, gate_fc_w, gate_fc_b,
           exp_conv1_w, exp_conv1_b, exp_conv2_w, exp_conv2_b, exp_hm_w,
           exp_hm_b, exp_wh_w, exp_wh_b, exp_reg_w, exp_reg_b):
    b, c, t, h, w = x.shape
    xr = x.reshape(b, c * t, h, w)

    scale = gate_bn_gamma / jnp.sqrt(1.0 + 1e-5)
    wfold = gate_conv_w * scale[:, None, None, None]
    Wgp = jnp.pad(wfold, ((0, 0), (0, 0), (1, 0), (1, 0)))
    Wg = Wgp.reshape(64, 15, 4, 2, 4, 2).transpose(2, 4, 3, 5, 1, 0).reshape(960, 64)
    beta2 = gate_bn_beta.reshape(1, 64)
    fct = jnp.pad(gate_fc_w.T, ((0, 0), (0, 125)))
    fcb = jnp.pad(gate_fc_b, ((0, 125))).reshape(1, 128)

    xg = pl.pallas_call(
        _s2d2_kernel,
        grid=(b, 48),
        in_specs=[pl.BlockSpec((1, 15, 8, 384), lambda i, j: (i, 0, j, 0))],
        out_specs=pl.BlockSpec((1, 4, 192, 60), lambda i, j: (i, j, 0, 0)),
        out_shape=jax.ShapeDtypeStruct((b, 192, 192, 60), jnp.float32),
    )(xr)
    xg_pad = jnp.pad(xg, ((0, 0), (2, 1), (2, 1), (0, 0)))

    xe = pl.pallas_call(
        _s2d4_kernel,
        grid=(b, 48),
        in_specs=[pl.BlockSpec((1, 15, 8, 384), lambda i, j: (i, 0, j, 0))],
        out_specs=pl.BlockSpec((1, 2, 96, 240), lambda i, j: (i, j, 0, 0)),
        out_shape=jax.ShapeDtypeStruct((b, 96, 96, 240), jnp.float32),
    )(xr)
    xe_pad = jnp.pad(xe, ((0, 0), (1, 0), (1, 0), (0, 0)))

    W1p = jnp.pad(exp_conv1_w, ((0, 0), (0, 0), (0, 0), (1, 0), (1, 0)))
    W1s = W1p.reshape(_E, 64, 15, 2, 4, 2, 4).transpose(0, 3, 5, 4, 6, 2, 1).reshape(_E, 960, 64)
    W2s = exp_conv2_w.transpose(0, 3, 4, 2, 1).reshape(_E, 576, 128)
    Whead = jnp.concatenate([exp_hm_w, exp_wh_w, exp_reg_w], axis=1)
    Whead = jnp.pad(Whead.reshape(_E, 5, 128).transpose(0, 2, 1), ((0, 0), (0, 0), (0, 3)))
    bhead = jnp.pad(jnp.concatenate([exp_hm_b, exp_wh_b, exp_reg_b], axis=1), ((0, 0), (0, 3)))

    logits = pl.pallas_call(
        _gate_kernel,
        grid=(b,),
        in_specs=[
            pl.BlockSpec(memory_space=pl.ANY),
            pl.BlockSpec((960, 64), lambda i: (0, 0)),
            pl.BlockSpec((1, 64), lambda i: (0, 0)),
            pl.BlockSpec((64, 128), lambda i: (0, 0)),
            pl.BlockSpec((1, 128), lambda i: (0, 0)),
        ],
        out_specs=pl.BlockSpec((1, 1, 128), lambda i: (i, 0, 0)),
        out_shape=jax.ShapeDtypeStruct((b, 1, 128), jnp.float32),
        scratch_shapes=[
            pltpu.VMEM((2, _GCH + 3, 195, 60), jnp.float32),
            pltpu.VMEM((194, 192, 64), jnp.float32),
            pltpu.SemaphoreType.DMA((2,)),
        ],
    )(xg_pad, Wg, beta2, fct, fcb)

    idx, aux = pl.pallas_call(
        _route_kernel,
        out_shape=(jax.ShapeDtypeStruct((b, 2), jnp.int32),
                   jax.ShapeDtypeStruct((1, 1), jnp.float32)),
    )(logits)
    idx_flat = idx.reshape(b * 2)

    grid_spec = pltpu.PrefetchScalarGridSpec(
        num_scalar_prefetch=1,
        grid=(b, 2),
        in_specs=[
            pl.BlockSpec((1, 97, 97, 240), lambda i, k, idx_r: (i, 0, 0, 0)),
            pl.BlockSpec((1, 960, 64), lambda i, k, idx_r: (idx_r[2 * i + k], 0, 0)),
            pl.BlockSpec((1, 1, 64), lambda i, k, idx_r: (idx_r[2 * i + k], 0, 0)),
            pl.BlockSpec((1, 576, 128), lambda i, k, idx_r: (idx_r[2 * i + k], 0, 0)),
            pl.BlockSpec((1, 1, 128), lambda i, k, idx_r: (idx_r[2 * i + k], 0, 0)),
            pl.BlockSpec((1, 128, 8), lambda i, k, idx_r: (idx_r[2 * i + k], 0, 0)),
            pl.BlockSpec((1, 1, 8), lambda i, k, idx_r: (idx_r[2 * i + k], 0, 0)),
        ],
        out_specs=pl.BlockSpec((1, 96, 96, 8), lambda i, k, idx_r: (i, 0, 0, 0)),
        scratch_shapes=[pltpu.VMEM((98, 98, 64), jnp.float32)],
    )
    out = pl.pallas_call(
        _expert_kernel,
        grid_spec=grid_spec,
        out_shape=jax.ShapeDtypeStruct((b, 96, 96, 8), jnp.float32),
    )(idx_flat, xe_pad, W1s, exp_conv1_b.reshape(_E, 1, 64), W2s,
      exp_conv2_b.reshape(_E, 1, 128), Whead, bhead.reshape(_E, 1, 8))

    o = out.transpose(0, 3, 1, 2)
    hm = o[:, 0:1]
    wh_o = o[:, 1:3]
    reg = o[:, 3:5]
    return hm, wh_o, reg, aux[0, 0]
